# Initial kernel scaffold; baseline (speedup 1.0000x reference)
import jax
import jax.numpy as jnp
from jax.experimental import pallas as pl


def kernel(x, edge_index, W, b):
    raise NotImplementedError("write your pallas kernel here")


# speedup vs baseline: 20.6228x; 20.6228x over previous
import functools

import jax
import jax.numpy as jnp
from jax import lax
from jax.experimental import pallas as pl
from jax.experimental.pallas import tpu as pltpu
from jax.experimental.pallas import tpu_sc as plsc

N = 10000
E = 320000
D = 128
C = 64

NC = 2
NS = 16
NW = NC * NS
PT = 640
NPAD = NS * PT
EPW = E // NW
CH = 125
NCHUNK = EPW // CH
EROWS = E // CH

_mesh = plsc.VectorSubcoreMesh(core_axis_name="c", subcore_axis_name="s")


def _deg_body(col2_hbm, ones_hbm, zeros_hbm, out_hbm, cidx_v, ones_v, deg_sh):
    c = lax.axis_index("c")
    s = lax.axis_index("s")
    wid = c * NS + s
    pltpu.sync_copy(ones_hbm, ones_v)
    pltpu.sync_copy(zeros_hbm, deg_sh.at[pl.ds(s * PT, PT)])
    pltpu.sync_copy(col2_hbm.at[pl.ds(wid * NCHUNK, NCHUNK), :], cidx_v)
    plsc.subcore_barrier()

    def body(j, carry):
        pltpu.sync_copy(ones_v, deg_sh.at[cidx_v.at[j]], add=True)
        return carry

    lax.fori_loop(0, NCHUNK, body, 0)
    plsc.subcore_barrier()
    pltpu.sync_copy(deg_sh.at[pl.ds(s * PT, PT)],
                    out_hbm.at[c, pl.ds(s * PT, PT), :])


_deg_call = functools.partial(
    pl.kernel,
    out_type=jax.ShapeDtypeStruct((NC, NPAD, 1), jnp.float32),
    mesh=_mesh,
    scratch_types=[
        pltpu.VMEM((NCHUNK, CH), jnp.int32),
        pltpu.VMEM((CH, 1), jnp.float32),
        pltpu.VMEM_SHARED((NPAD, 1), jnp.float32),
    ],
)(_deg_body)


def _prop_body(g_hbm, row2_hbm, col2_hbm, zeros_hbm, out_hbm,
               ridx_v, cidx_v, rows_v, acc_sh, gsem):
    c = lax.axis_index("c")
    s = lax.axis_index("s")
    wid = c * NS + s
    pltpu.sync_copy(zeros_hbm, acc_sh.at[pl.ds(s * PT, PT)])
    pltpu.sync_copy(row2_hbm.at[pl.ds(wid * NCHUNK, NCHUNK), :], ridx_v)
    pltpu.sync_copy(col2_hbm.at[pl.ds(wid * NCHUNK, NCHUNK), :], cidx_v)
    plsc.subcore_barrier()

    def body(j, carry):
        pltpu.async_copy(g_hbm.at[ridx_v.at[j]], rows_v, gsem).wait()
        pltpu.sync_copy(rows_v, acc_sh.at[cidx_v.at[j]], add=True)
        return carry

    lax.fori_loop(0, NCHUNK, body, 0)
    plsc.subcore_barrier()
    pltpu.sync_copy(acc_sh.at[pl.ds(s * PT, PT)],
                    out_hbm.at[c, pl.ds(s * PT, PT), :])


_prop_call = functools.partial(
    pl.kernel,
    out_type=jax.ShapeDtypeStruct((NC, NPAD, D), jnp.float32),
    mesh=_mesh,
    scratch_types=[
        pltpu.VMEM((NCHUNK, CH), jnp.int32),
        pltpu.VMEM((NCHUNK, CH), jnp.int32),
        pltpu.VMEM((CH, D), jnp.float32),
        pltpu.VMEM_SHARED((NPAD, D), jnp.float32),
        pltpu.SemaphoreType.DMA,
    ],
)(_prop_body)


RB = 1024
GRID = (NPAD + RB - 1) // RB


def _scale0_body(dega_ref, degb_ref, x_ref, dinv_ref, g0_ref):
    deg = dega_ref[...] + degb_ref[...] + 1.0
    dinv = lax.rsqrt(deg)
    dinv_ref[...] = dinv
    g0_ref[...] = dinv * x_ref[...]


_scale0 = pl.pallas_call(
    _scale0_body,
    grid=(GRID,),
    in_specs=[
        pl.BlockSpec((RB, 1), lambda i: (i, 0)),
        pl.BlockSpec((RB, 1), lambda i: (i, 0)),
        pl.BlockSpec((RB, D), lambda i: (i, 0)),
    ],
    out_specs=[
        pl.BlockSpec((RB, 1), lambda i: (i, 0)),
        pl.BlockSpec((RB, D), lambda i: (i, 0)),
    ],
    out_shape=[
        jax.ShapeDtypeStruct((N, 1), jnp.float32),
        jax.ShapeDtypeStruct((N, D), jnp.float32),
    ],
)


def _scale1_body(dinv_ref, pa_ref, pb_ref, g0_ref, g2_ref):
    dinv = dinv_ref[...]
    h = pa_ref[...] + pb_ref[...] + g0_ref[...]
    g2_ref[...] = h * (dinv * dinv)


_scale1 = pl.pallas_call(
    _scale1_body,
    grid=(GRID,),
    in_specs=[
        pl.BlockSpec((RB, 1), lambda i: (i, 0)),
        pl.BlockSpec((RB, D), lambda i: (i, 0)),
        pl.BlockSpec((RB, D), lambda i: (i, 0)),
        pl.BlockSpec((RB, D), lambda i: (i, 0)),
    ],
    out_specs=pl.BlockSpec((RB, D), lambda i: (i, 0)),
    out_shape=jax.ShapeDtypeStruct((N, D), jnp.float32),
)


def _final_body(dinv_ref, pa_ref, pb_ref, g2_ref, w_ref, b_ref, out_ref):
    h = dinv_ref[...] * (pa_ref[...] + pb_ref[...] + g2_ref[...])
    out_ref[...] = (
        jnp.dot(h, w_ref[...].T, preferred_element_type=jnp.float32)
        + b_ref[...]
    )


_final = pl.pallas_call(
    _final_body,
    grid=(GRID,),
    in_specs=[
        pl.BlockSpec((RB, 1), lambda i: (i, 0)),
        pl.BlockSpec((RB, D), lambda i: (i, 0)),
        pl.BlockSpec((RB, D), lambda i: (i, 0)),
        pl.BlockSpec((RB, D), lambda i: (i, 0)),
        pl.BlockSpec((C, D), lambda i: (0, 0)),
        pl.BlockSpec((1, C), lambda i: (0, 0)),
    ],
    out_specs=pl.BlockSpec((RB, C), lambda i: (i, 0)),
    out_shape=jax.ShapeDtypeStruct((N, C), jnp.float32),
)


def kernel(x, edge_index, W, b):
    ei = edge_index.astype(jnp.int32)
    row2 = ei[0].reshape(EROWS, CH)
    col2 = ei[1].reshape(EROWS, CH)
    ones_ch = jnp.ones((CH, 1), jnp.float32)
    zeros_1d = jnp.zeros((PT, 1), jnp.float32)
    zeros_2d = jnp.zeros((PT, D), jnp.float32)

    deg_parts = _deg_call(col2, ones_ch, zeros_1d)
    dega = deg_parts[0]
    degb = deg_parts[1]
    dinv, g0 = _scale0(dega, degb, x)

    p = _prop_call(g0, row2, col2, zeros_2d)
    g2 = _scale1(dinv, p[0], p[1], g0)
    q = _prop_call(g2, row2, col2, zeros_2d)
    return _final(dinv, q[0], q[1], g2, W, b.reshape(1, C))
